# baseline (speedup 1.0000x reference)
import functools
import jax
import jax.numpy as jnp
from jax import lax
from jax.experimental import pallas as pl
from jax.experimental.pallas import tpu as pltpu
from jax.experimental.pallas import tpu_sc as plsc

N = 10000
E = 320000
NC = 2
NS = 16
K = 80

_MESH = dict(
    mesh=plsc.VectorSubcoreMesh(
        core_axis_name="c", subcore_axis_name="s", num_cores=NC, num_subcores=NS
    )
)


@functools.partial(
    pl.kernel,
    out_type=jax.ShapeDtypeStruct((2, 1, N), jnp.float32),
    scratch_types=[
        pltpu.VMEM((250, K), jnp.int32),
        pltpu.VMEM((K,), jnp.float32),
        pltpu.VMEM((2000,), jnp.float32),
        pltpu.VMEM_SHARED((N,), jnp.float32),
        pltpu.SemaphoreType.DMA,
    ],
    **_MESH,
)
def _sc_degrees(eidx, out, idx_all, ones_v, zbuf, acc, sem):
    cid = lax.axis_index("c")
    sid = lax.axis_index("s")

    for j in range(K // 16):
        ones_v[pl.ds(j * 16, 16)] = jnp.ones((16,), jnp.float32)

    @pl.when(sid == 0)
    def _zero():
        def zrow(i, _):
            zbuf[pl.ds(i * 16, 16)] = jnp.zeros((16,), jnp.float32)
            return 0
        lax.fori_loop(0, 2000 // 16, zrow, 0)
        for j in range(N // 2000):
            pltpu.sync_copy(zbuf, acc.at[pl.ds(j * 2000, 2000)])

    pltpu.sync_copy(eidx.at[cid, sid], idx_all)

    plsc.subcore_barrier()

    def outer(o, _):
        ds = []
        for b in range(10):
            g = o * 10 + b
            ds.append(pltpu.async_copy(ones_v, acc.at[idx_all.at[g]], sem, add=True))
        for d in ds:
            d.wait()
        return 0

    lax.fori_loop(0, 25, outer, 0)

    plsc.subcore_barrier()

    @pl.when(sid == 0)
    def _writeout():
        pltpu.sync_copy(acc, out.at[cid, 0])


_PER_W = E // (NC * NS)
KA = 128
NCH = 80
EPW = NCH * KA
NA = N + 8
ISB = 16
NBLK = NCH // ISB


@functools.partial(
    pl.kernel,
    out_type=jax.ShapeDtypeStruct((NC, N, 128), jnp.float32),
    scratch_types=[
        pltpu.VMEM((2, ISB, 128), jnp.int32),
        pltpu.VMEM((NCH, 128), jnp.int32),
        pltpu.VMEM((2, KA, 128), jnp.float32),
        pltpu.VMEM_SHARED((NA, 128), jnp.float32),
        pltpu.SemaphoreType.DMA((2,)),
        pltpu.SemaphoreType.DMA((2,)),
        pltpu.SemaphoreType.DMA,
    ],
    **_MESH,
)
def _sc_agg(table, sidx, didx, out, isb, idx_d, rows, acc, sem_i, sem_g,
            sem_s):
    cid = lax.axis_index("c")
    sid = lax.axis_index("s")
    w = cid * NS + sid

    def zrow(i, _):
        for j in range(8):
            rows[0, i, pl.ds(j * 16, 16)] = jnp.zeros((16,), jnp.float32)
        return 0
    lax.fori_loop(0, 40, zrow, 0)

    @pl.when(sid < 10)
    def _zero():
        def zblk(j, _):
            pltpu.sync_copy(rows.at[0, pl.ds(0, 40)],
                            acc.at[pl.ds(sid * 1000 + j * 40, 40)])
            return 0
        lax.fori_loop(0, 25, zblk, 0)

    pltpu.sync_copy(didx.at[w], idx_d)
    pltpu.sync_copy(sidx.at[w, pl.ds(0, ISB)], isb.at[0])

    def isb_start(o):
        pltpu.async_copy(sidx.at[w, pl.ds(o * ISB, ISB)], isb.at[o % 2],
                         sem_i.at[o % 2])

    def isb_wait(o):
        pltpu.make_async_copy(sidx.at[w, pl.ds(o * ISB, ISB)], isb.at[o % 2],
                              sem_i.at[o % 2]).wait()

    def gather_start(g):
        pltpu.async_copy(table.at[isb.at[(g // ISB) % 2, g % ISB]],
                         rows.at[g % 2], sem_g.at[g % 2])

    def gather_wait(g):
        pltpu.make_async_copy(table.at[isb.at[(g // ISB) % 2, g % ISB]],
                              rows.at[g % 2], sem_g.at[g % 2]).wait()

    isb_start(1)

    plsc.subcore_barrier()

    gather_start(0)
    gather_start(1)

    for g in range(NCH):
        o, b = divmod(g, ISB)
        if b == ISB - 2 and o + 1 < NBLK:
            isb_wait(o + 1)
        gather_wait(g)
        pltpu.async_copy(rows.at[g % 2], acc.at[idx_d.at[g]], sem_s,
                         add=True).wait()
        if g + 2 < NCH:
            gather_start(g + 2)
        if b == ISB - 1 and o + 2 < NBLK:
            isb_start(o + 2)

    plsc.subcore_barrier()

    @pl.when(sid < 10)
    def _writeout():
        pltpu.sync_copy(
            acc.at[pl.ds(sid * 1000, 1000)],
            out.at[cid, pl.ds(sid * 1000, 1000)],
        )


_BR = 400


def _prep_body(x_ref, ds_ref, dd_ref, h0_ref, ns_ref, nd_ref):
    ns = lax.rsqrt(jnp.maximum(ds_ref[...], 1.0))
    nd = lax.rsqrt(jnp.maximum(dd_ref[...], 1.0))
    h0_ref[...] = x_ref[...] * ns
    ns_ref[...] = ns
    nd_ref[...] = nd


def _prep(x, ds_col, dd_col):
    return pl.pallas_call(
        _prep_body,
        grid=(N // _BR,),
        in_specs=[
            pl.BlockSpec((_BR, 128), lambda i: (i, 0)),
            pl.BlockSpec((_BR, 1), lambda i: (i, 0)),
            pl.BlockSpec((_BR, 1), lambda i: (i, 0)),
        ],
        out_specs=[
            pl.BlockSpec((_BR, 128), lambda i: (i, 0)),
            pl.BlockSpec((_BR, 1), lambda i: (i, 0)),
            pl.BlockSpec((_BR, 1), lambda i: (i, 0)),
        ],
        out_shape=[
            jax.ShapeDtypeStruct((N, 128), jnp.float32),
            jax.ShapeDtypeStruct((N, 1), jnp.float32),
            jax.ShapeDtypeStruct((N, 1), jnp.float32),
        ],
    )(x, ds_col, dd_col)


def _mid_body(p0_ref, p1_ref, nd_ref, ns_ref, w1_ref, b1_ref, w2_ref, o_ref):
    agg = (p0_ref[...] + p1_ref[...]) * nd_ref[...]
    h = jnp.dot(agg, w1_ref[...], preferred_element_type=jnp.float32)
    h = jnp.maximum(h + b1_ref[...], 0.0)
    h2 = jnp.dot(h, w2_ref[...], preferred_element_type=jnp.float32)
    o_ref[...] = h2 * ns_ref[...]


def _mid(p0, p1, nd, ns, W1, b1r, W2):
    return pl.pallas_call(
        _mid_body,
        grid=(N // _BR,),
        in_specs=[
            pl.BlockSpec((_BR, 128), lambda i: (i, 0)),
            pl.BlockSpec((_BR, 128), lambda i: (i, 0)),
            pl.BlockSpec((_BR, 1), lambda i: (i, 0)),
            pl.BlockSpec((_BR, 1), lambda i: (i, 0)),
            pl.BlockSpec((128, 256), lambda i: (0, 0)),
            pl.BlockSpec((1, 256), lambda i: (0, 0)),
            pl.BlockSpec((256, 128), lambda i: (0, 0)),
        ],
        out_specs=pl.BlockSpec((_BR, 128), lambda i: (i, 0)),
        out_shape=jax.ShapeDtypeStruct((N, 128), jnp.float32),
    )(p0, p1, nd, ns, W1, b1r, W2)


def _post_body(p0_ref, p1_ref, nd_ref, b2_ref, o_ref):
    agg = (p0_ref[...] + p1_ref[...]) * nd_ref[...]
    o_ref[...] = jnp.maximum(agg + b2_ref[...], 0.0)


def _post(p0, p1, nd, b2r):
    return pl.pallas_call(
        _post_body,
        grid=(N // _BR,),
        in_specs=[
            pl.BlockSpec((_BR, 128), lambda i: (i, 0)),
            pl.BlockSpec((_BR, 128), lambda i: (i, 0)),
            pl.BlockSpec((_BR, 1), lambda i: (i, 0)),
            pl.BlockSpec((1, 128), lambda i: (0, 0)),
        ],
        out_specs=pl.BlockSpec((_BR, 128), lambda i: (i, 0)),
        out_shape=jax.ShapeDtypeStruct((N, 128), jnp.float32),
    )(p0, p1, nd, b2r)


def kernel(x, edge_index, W1, b1, W2, b2):
    eidx = edge_index.astype(jnp.int32)
    npad = EPW - _PER_W
    src_p = jnp.concatenate(
        [eidx[0].reshape(NC * NS, _PER_W),
         jnp.zeros((NC * NS, npad), jnp.int32)], axis=1
    ).reshape(NC * NS, NCH, KA)
    dst_p = jnp.concatenate(
        [eidx[1].reshape(NC * NS, _PER_W),
         jnp.full((NC * NS, npad), N, jnp.int32)], axis=1
    ).reshape(NC * NS, NCH, KA)

    deg = _sc_degrees(eidx.reshape(2, NS, E // NS // K, K))
    ds_col = deg[0, 0].reshape(N, 1)
    dd_col = deg[1, 0].reshape(N, 1)

    h0, ns, nd = _prep(x, ds_col, dd_col)

    p = _sc_agg(h0, src_p, dst_p)
    h1s = _mid(p[0], p[1], nd, ns, W1, b1.reshape(1, -1), W2)

    q = _sc_agg(h1s, src_p, dst_p)
    return _post(q[0], q[1], nd, b2.reshape(1, -1))


# speedup vs baseline: 2.8183x; 2.8183x over previous
import functools
import jax
import jax.numpy as jnp
from jax import lax
from jax.experimental import pallas as pl
from jax.experimental.pallas import tpu as pltpu
from jax.experimental.pallas import tpu_sc as plsc

N = 10000
E = 320000
NC = 2
NS = 16
K = 80

_MESH = dict(
    mesh=plsc.VectorSubcoreMesh(
        core_axis_name="c", subcore_axis_name="s", num_cores=NC, num_subcores=NS
    )
)


@functools.partial(
    pl.kernel,
    out_type=jax.ShapeDtypeStruct((2, 1, N), jnp.float32),
    scratch_types=[
        pltpu.VMEM((250, K), jnp.int32),
        pltpu.VMEM((K,), jnp.float32),
        pltpu.VMEM((2000,), jnp.float32),
        pltpu.VMEM_SHARED((N,), jnp.float32),
        pltpu.SemaphoreType.DMA,
    ],
    **_MESH,
)
def _sc_degrees(eidx, out, idx_all, ones_v, zbuf, acc, sem):
    cid = lax.axis_index("c")
    sid = lax.axis_index("s")

    for j in range(K // 16):
        ones_v[pl.ds(j * 16, 16)] = jnp.ones((16,), jnp.float32)

    @pl.when(sid == 0)
    def _zero():
        def zrow(i, _):
            zbuf[pl.ds(i * 16, 16)] = jnp.zeros((16,), jnp.float32)
            return 0
        lax.fori_loop(0, 2000 // 16, zrow, 0)
        for j in range(N // 2000):
            pltpu.sync_copy(zbuf, acc.at[pl.ds(j * 2000, 2000)])

    pltpu.sync_copy(eidx.at[cid, sid], idx_all)

    plsc.subcore_barrier()

    def outer(o, _):
        ds = []
        for b in range(10):
            g = o * 10 + b
            ds.append(pltpu.async_copy(ones_v, acc.at[idx_all.at[g]], sem, add=True))
        for d in ds:
            d.wait()
        return 0

    lax.fori_loop(0, 25, outer, 0)

    plsc.subcore_barrier()

    @pl.when(sid == 0)
    def _writeout():
        pltpu.sync_copy(acc, out.at[cid, 0])


_NB = 2
_NCHUNK = E // (NC * NS) // K
_PER_W = E // (NC * NS)


@functools.partial(
    pl.kernel,
    out_type=jax.ShapeDtypeStruct((NC, N, 128), jnp.float32),
    scratch_types=[
        pltpu.VMEM((_PER_W,), jnp.int32),
        pltpu.VMEM((_NCHUNK, K), jnp.int32),
        pltpu.VMEM((_NB, K, 128), jnp.float32),
        pltpu.VMEM_SHARED((N, 128), jnp.float32),
        pltpu.SemaphoreType.DMA((_NB,)),
        pltpu.SemaphoreType.DMA((_NB,)),
    ],
    **_MESH,
)
def _sc_agg(table, sidx, didx, out, idx_s, idx_d, rows, acc, sem_g, sem_s):
    cid = lax.axis_index("c")
    sid = lax.axis_index("s")
    w = cid * NS + sid

    def zrow(i, _):
        for j in range(8):
            rows[0, i, pl.ds(j * 16, 16)] = jnp.zeros((16,), jnp.float32)
        return 0
    lax.fori_loop(0, 40, zrow, 0)

    @pl.when(sid < 10)
    def _zero():
        def zblk(j, _):
            pltpu.sync_copy(rows.at[0, pl.ds(0, 40)],
                            acc.at[pl.ds(sid * 1000 + j * 40, 40)])
            return 0
        lax.fori_loop(0, 25, zblk, 0)

    pltpu.sync_copy(sidx.at[w], idx_s)
    pltpu.sync_copy(didx.at[w], idx_d)

    plsc.subcore_barrier()

    def gather_start(g, b):
        pltpu.async_copy(table.at[idx_s.at[pl.ds(g * K, K)]], rows.at[b],
                         sem_g.at[b])

    def gather_wait(g, b):
        pltpu.make_async_copy(table.at[idx_s.at[pl.ds(g * K, K)]], rows.at[b],
                              sem_g.at[b]).wait()

    def step(g, b):
        gather_wait(g, b)
        sc = pltpu.async_copy(rows.at[b], acc.at[idx_d.at[g]], sem_s.at[b],
                              add=True)
        sc.wait()

        @pl.when(g + _NB < _NCHUNK)
        def _pref():
            gather_start(g + _NB, b)

    for b in range(_NB):
        gather_start(b, b)

    def outer(o, _):
        for b in range(_NB):
            step(o * _NB + b, b)
        return 0

    lax.fori_loop(0, (_NCHUNK - 1) // _NB, outer, 0)
    step(_NCHUNK - 1, (_NCHUNK - 1) % _NB)

    plsc.subcore_barrier()

    @pl.when(sid < 10)
    def _writeout():
        pltpu.sync_copy(
            acc.at[pl.ds(sid * 1000, 1000)],
            out.at[cid, pl.ds(sid * 1000, 1000)],
        )


_BR = 2000


def _prep_body(x_ref, ds_ref, h0_ref):
    ns = lax.rsqrt(jnp.maximum(ds_ref[...], 1.0))
    h0_ref[...] = x_ref[...] * ns


def _prep(x, ds_col):
    return pl.pallas_call(
        _prep_body,
        grid=(N // _BR,),
        in_specs=[
            pl.BlockSpec((_BR, 128), lambda i: (i, 0)),
            pl.BlockSpec((_BR, 1), lambda i: (i, 0)),
        ],
        out_specs=pl.BlockSpec((_BR, 128), lambda i: (i, 0)),
        out_shape=jax.ShapeDtypeStruct((N, 128), jnp.float32),
    )(x, ds_col)


def _mid_body(p0_ref, p1_ref, dd_ref, ds_ref, w1_ref, b1_ref, w2_ref, o_ref):
    nd = lax.rsqrt(jnp.maximum(dd_ref[...], 1.0))
    ns = lax.rsqrt(jnp.maximum(ds_ref[...], 1.0))
    agg = (p0_ref[...] + p1_ref[...]) * nd
    h = jnp.dot(agg, w1_ref[...], preferred_element_type=jnp.float32)
    h = jnp.maximum(h + b1_ref[...], 0.0)
    h2 = jnp.dot(h, w2_ref[...], preferred_element_type=jnp.float32)
    o_ref[...] = h2 * ns


def _mid(p0, p1, dd_col, ds_col, W1, b1r, W2):
    return pl.pallas_call(
        _mid_body,
        grid=(N // _BR,),
        in_specs=[
            pl.BlockSpec((_BR, 128), lambda i: (i, 0)),
            pl.BlockSpec((_BR, 128), lambda i: (i, 0)),
            pl.BlockSpec((_BR, 1), lambda i: (i, 0)),
            pl.BlockSpec((_BR, 1), lambda i: (i, 0)),
            pl.BlockSpec((128, 256), lambda i: (0, 0)),
            pl.BlockSpec((1, 256), lambda i: (0, 0)),
            pl.BlockSpec((256, 128), lambda i: (0, 0)),
        ],
        out_specs=pl.BlockSpec((_BR, 128), lambda i: (i, 0)),
        out_shape=jax.ShapeDtypeStruct((N, 128), jnp.float32),
    )(p0, p1, dd_col, ds_col, W1, b1r, W2)


def _post_body(p0_ref, p1_ref, dd_ref, b2_ref, o_ref):
    nd = lax.rsqrt(jnp.maximum(dd_ref[...], 1.0))
    agg = (p0_ref[...] + p1_ref[...]) * nd
    o_ref[...] = jnp.maximum(agg + b2_ref[...], 0.0)


def _post(p0, p1, dd_col, b2r):
    return pl.pallas_call(
        _post_body,
        grid=(N // _BR,),
        in_specs=[
            pl.BlockSpec((_BR, 128), lambda i: (i, 0)),
            pl.BlockSpec((_BR, 128), lambda i: (i, 0)),
            pl.BlockSpec((_BR, 1), lambda i: (i, 0)),
            pl.BlockSpec((1, 128), lambda i: (0, 0)),
        ],
        out_specs=pl.BlockSpec((_BR, 128), lambda i: (i, 0)),
        out_shape=jax.ShapeDtypeStruct((N, 128), jnp.float32),
    )(p0, p1, dd_col, b2r)


def kernel(x, edge_index, W1, b1, W2, b2):
    eidx = edge_index.astype(jnp.int32)
    src2 = eidx[0].reshape(NC * NS, _PER_W)
    dst3 = eidx[1].reshape(NC * NS, _NCHUNK, K)

    deg = _sc_degrees(eidx.reshape(2, NS, E // NS // K, K))
    ds_col = deg[0, 0].reshape(N, 1)
    dd_col = deg[1, 0].reshape(N, 1)

    h0 = _prep(x, ds_col)

    p = _sc_agg(h0, src2, dst3)
    h1s = _mid(p[0], p[1], dd_col, ds_col, W1, b1.reshape(1, -1), W2)

    q = _sc_agg(h1s, src2, dst3)
    return _post(q[0], q[1], dd_col, b2.reshape(1, -1))


# speedup vs baseline: 3.1863x; 1.1306x over previous
import functools
import jax
import jax.numpy as jnp
from jax import lax
from jax.experimental import pallas as pl
from jax.experimental.pallas import tpu as pltpu
from jax.experimental.pallas import tpu_sc as plsc

N = 10000
E = 320000
NC = 2
NS = 16
K = 80

_MESH = dict(
    mesh=plsc.VectorSubcoreMesh(
        core_axis_name="c", subcore_axis_name="s", num_cores=NC, num_subcores=NS
    )
)


@functools.partial(
    pl.kernel,
    out_type=jax.ShapeDtypeStruct((2, 1, N), jnp.float32),
    scratch_types=[
        pltpu.VMEM((250, K), jnp.int32),
        pltpu.VMEM((K,), jnp.float32),
        pltpu.VMEM((2000,), jnp.float32),
        pltpu.VMEM_SHARED((N,), jnp.float32),
        pltpu.SemaphoreType.DMA,
    ],
    **_MESH,
)
def _sc_degrees(eidx, out, idx_all, ones_v, zbuf, acc, sem):
    cid = lax.axis_index("c")
    sid = lax.axis_index("s")

    for j in range(K // 16):
        ones_v[pl.ds(j * 16, 16)] = jnp.ones((16,), jnp.float32)

    @pl.when(sid == 0)
    def _zero():
        def zrow(i, _):
            zbuf[pl.ds(i * 16, 16)] = jnp.zeros((16,), jnp.float32)
            return 0
        lax.fori_loop(0, 2000 // 16, zrow, 0)
        for j in range(N // 2000):
            pltpu.sync_copy(zbuf, acc.at[pl.ds(j * 2000, 2000)])

    pltpu.sync_copy(eidx.at[cid, sid], idx_all)

    plsc.subcore_barrier()

    def outer(o, _):
        ds = []
        for b in range(10):
            g = o * 10 + b
            ds.append(pltpu.async_copy(ones_v, acc.at[idx_all.at[g]], sem, add=True))
        for d in ds:
            d.wait()
        return 0

    lax.fori_loop(0, 25, outer, 0)

    plsc.subcore_barrier()

    @pl.when(sid == 0)
    def _writeout():
        pltpu.sync_copy(acc, out.at[cid, 0])


_NB = 3
_NCHUNK = E // (NC * NS) // K
_PER_W = E // (NC * NS)


@functools.partial(
    pl.kernel,
    out_type=jax.ShapeDtypeStruct((NC, N, 128), jnp.float32),
    scratch_types=[
        pltpu.VMEM((_PER_W,), jnp.int32),
        pltpu.VMEM((_PER_W,), jnp.int32),
        pltpu.VMEM((_NB, K, 128), jnp.float32),
        pltpu.VMEM_SHARED((N, 128), jnp.float32),
        pltpu.SemaphoreType.DMA((_NB,)),
        pltpu.SemaphoreType.DMA((_NB,)),
    ],
    **_MESH,
)
def _sc_agg(table, sidx, didx, out, idx_s, idx_d, rows, acc, sem_g, sem_s):
    cid = lax.axis_index("c")
    sid = lax.axis_index("s")
    w = cid * NS + sid

    def zrow(i, _):
        for j in range(8):
            rows[0, i, pl.ds(j * 16, 16)] = jnp.zeros((16,), jnp.float32)
        return 0
    lax.fori_loop(0, 40, zrow, 0)

    @pl.when(sid < 10)
    def _zero():
        def zblk(j, _):
            pltpu.sync_copy(rows.at[0, pl.ds(0, 40)],
                            acc.at[pl.ds(sid * 1000 + j * 40, 40)])
            return 0
        lax.fori_loop(0, 25, zblk, 0)

    pltpu.sync_copy(sidx.at[w], idx_s)
    pltpu.sync_copy(didx.at[w], idx_d)

    plsc.subcore_barrier()

    def gather_start(g, b):
        pltpu.async_copy(table.at[idx_s.at[pl.ds(g * K, K)]], rows.at[b],
                         sem_g.at[b])

    def gather_wait(g, b):
        pltpu.make_async_copy(table.at[idx_s.at[pl.ds(g * K, K)]], rows.at[b],
                              sem_g.at[b]).wait()

    def scatter_start(g, b):
        pltpu.async_copy(rows.at[b], acc.at[idx_d.at[pl.ds(g * K, K)]],
                         sem_s.at[b], add=True)

    def scatter_wait(g, b):
        pltpu.make_async_copy(rows.at[b], acc.at[idx_d.at[pl.ds(g * K, K)]],
                              sem_s.at[b]).wait()

    def step(g, b):
        gather_wait(g, b)
        scatter_start(g, b)

        @pl.when(g >= 1)
        def _ret():
            scatter_wait(g - 1, (b - 1) % _NB)

        @pl.when(g + 2 < _NCHUNK)
        def _pref():
            gather_start(g + 2, (b + 2) % _NB)

    gather_start(0, 0)
    gather_start(1, 1)

    def outer(o, _):
        for b in range(_NB):
            step(o * _NB + b, b)
        return 0

    lax.fori_loop(0, (_NCHUNK - 2) // _NB, outer, 0)
    for g in range(((_NCHUNK - 2) // _NB) * _NB, _NCHUNK):
        step(g, g % _NB)
    scatter_wait(_NCHUNK - 1, (_NCHUNK - 1) % _NB)

    plsc.subcore_barrier()

    @pl.when(sid < 10)
    def _writeout():
        pltpu.sync_copy(
            acc.at[pl.ds(sid * 1000, 1000)],
            out.at[cid, pl.ds(sid * 1000, 1000)],
        )


_BR = 2000


def _prep_body(x_ref, ds_ref, h0_ref):
    ns = lax.rsqrt(jnp.maximum(ds_ref[...], 1.0))
    h0_ref[...] = x_ref[...] * ns


def _prep(x, ds_col):
    return pl.pallas_call(
        _prep_body,
        grid=(N // _BR,),
        in_specs=[
            pl.BlockSpec((_BR, 128), lambda i: (i, 0)),
            pl.BlockSpec((_BR, 1), lambda i: (i, 0)),
        ],
        out_specs=pl.BlockSpec((_BR, 128), lambda i: (i, 0)),
        out_shape=jax.ShapeDtypeStruct((N, 128), jnp.float32),
    )(x, ds_col)


def _mid_body(p0_ref, p1_ref, dd_ref, ds_ref, w1_ref, b1_ref, w2_ref, o_ref):
    nd = lax.rsqrt(jnp.maximum(dd_ref[...], 1.0))
    ns = lax.rsqrt(jnp.maximum(ds_ref[...], 1.0))
    agg = (p0_ref[...] + p1_ref[...]) * nd
    h = jnp.dot(agg, w1_ref[...], preferred_element_type=jnp.float32)
    h = jnp.maximum(h + b1_ref[...], 0.0)
    h2 = jnp.dot(h, w2_ref[...], preferred_element_type=jnp.float32)
    o_ref[...] = h2 * ns


def _mid(p0, p1, dd_col, ds_col, W1, b1r, W2):
    return pl.pallas_call(
        _mid_body,
        grid=(N // _BR,),
        in_specs=[
            pl.BlockSpec((_BR, 128), lambda i: (i, 0)),
            pl.BlockSpec((_BR, 128), lambda i: (i, 0)),
            pl.BlockSpec((_BR, 1), lambda i: (i, 0)),
            pl.BlockSpec((_BR, 1), lambda i: (i, 0)),
            pl.BlockSpec((128, 256), lambda i: (0, 0)),
            pl.BlockSpec((1, 256), lambda i: (0, 0)),
            pl.BlockSpec((256, 128), lambda i: (0, 0)),
        ],
        out_specs=pl.BlockSpec((_BR, 128), lambda i: (i, 0)),
        out_shape=jax.ShapeDtypeStruct((N, 128), jnp.float32),
    )(p0, p1, dd_col, ds_col, W1, b1r, W2)


def _post_body(p0_ref, p1_ref, dd_ref, b2_ref, o_ref):
    nd = lax.rsqrt(jnp.maximum(dd_ref[...], 1.0))
    agg = (p0_ref[...] + p1_ref[...]) * nd
    o_ref[...] = jnp.maximum(agg + b2_ref[...], 0.0)


def _post(p0, p1, dd_col, b2r):
    return pl.pallas_call(
        _post_body,
        grid=(N // _BR,),
        in_specs=[
            pl.BlockSpec((_BR, 128), lambda i: (i, 0)),
            pl.BlockSpec((_BR, 128), lambda i: (i, 0)),
            pl.BlockSpec((_BR, 1), lambda i: (i, 0)),
            pl.BlockSpec((1, 128), lambda i: (0, 0)),
        ],
        out_specs=pl.BlockSpec((_BR, 128), lambda i: (i, 0)),
        out_shape=jax.ShapeDtypeStruct((N, 128), jnp.float32),
    )(p0, p1, dd_col, b2r)


def kernel(x, edge_index, W1, b1, W2, b2):
    eidx = edge_index.astype(jnp.int32)
    src2 = eidx[0].reshape(NC * NS, _PER_W)
    dst3 = eidx[1].reshape(NC * NS, _PER_W)

    deg = _sc_degrees(eidx.reshape(2, NS, E // NS // K, K))
    ds_col = deg[0, 0].reshape(N, 1)
    dd_col = deg[1, 0].reshape(N, 1)

    h0 = _prep(x, ds_col)

    p = _sc_agg(h0, src2, dst3)
    h1s = _mid(p[0], p[1], dd_col, ds_col, W1, b1.reshape(1, -1), W2)

    q = _sc_agg(h1s, src2, dst3)
    return _post(q[0], q[1], dd_col, b2.reshape(1, -1))


# speedup vs baseline: 3.1905x; 1.0013x over previous
import functools
import jax
import jax.numpy as jnp
from jax import lax
from jax.experimental import pallas as pl
from jax.experimental.pallas import tpu as pltpu
from jax.experimental.pallas import tpu_sc as plsc

N = 10000
E = 320000
NC = 2
NS = 16
K = 80

_MESH = dict(
    mesh=plsc.VectorSubcoreMesh(
        core_axis_name="c", subcore_axis_name="s", num_cores=NC, num_subcores=NS
    )
)


@functools.partial(
    pl.kernel,
    out_type=jax.ShapeDtypeStruct((2, 1, N), jnp.float32),
    scratch_types=[
        pltpu.VMEM((250, K), jnp.int32),
        pltpu.VMEM((K,), jnp.float32),
        pltpu.VMEM((2000,), jnp.float32),
        pltpu.VMEM_SHARED((N,), jnp.float32),
        pltpu.SemaphoreType.DMA,
    ],
    **_MESH,
)
def _sc_degrees(eidx, out, idx_all, ones_v, zbuf, acc, sem):
    cid = lax.axis_index("c")
    sid = lax.axis_index("s")

    for j in range(K // 16):
        ones_v[pl.ds(j * 16, 16)] = jnp.ones((16,), jnp.float32)

    @pl.when(sid == 0)
    def _zero():
        def zrow(i, _):
            zbuf[pl.ds(i * 16, 16)] = jnp.zeros((16,), jnp.float32)
            return 0
        lax.fori_loop(0, 2000 // 16, zrow, 0)
        for j in range(N // 2000):
            pltpu.sync_copy(zbuf, acc.at[pl.ds(j * 2000, 2000)])

    pltpu.sync_copy(eidx.at[cid, sid], idx_all)

    plsc.subcore_barrier()

    def outer(o, _):
        ds = []
        for b in range(10):
            g = o * 10 + b
            ds.append(pltpu.async_copy(ones_v, acc.at[idx_all.at[g]], sem, add=True))
        for d in ds:
            d.wait()
        return 0

    lax.fori_loop(0, 25, outer, 0)

    plsc.subcore_barrier()

    @pl.when(sid == 0)
    def _writeout():
        pltpu.sync_copy(acc, out.at[cid, 0])


_NB = 3
_NCHUNK = E // (NC * NS) // K
_PER_W = E // (NC * NS)


@functools.partial(
    pl.kernel,
    out_type=jax.ShapeDtypeStruct((NC, N, 128), jnp.float32),
    scratch_types=[
        pltpu.VMEM((_PER_W,), jnp.int32),
        pltpu.VMEM((_PER_W,), jnp.int32),
        pltpu.VMEM((_NB, K, 128), jnp.float32),
        pltpu.VMEM_SHARED((N, 128), jnp.float32),
        pltpu.SemaphoreType.DMA((_NB,)),
        pltpu.SemaphoreType.DMA((_NB,)),
    ],
    **_MESH,
)
def _sc_agg(table, sidx, didx, out, idx_s, idx_d, rows, acc, sem_g, sem_s):
    cid = lax.axis_index("c")
    sid = lax.axis_index("s")
    w = cid * NS + sid

    def zrow(i, _):
        for j in range(8):
            rows[0, i, pl.ds(j * 16, 16)] = jnp.zeros((16,), jnp.float32)
        return 0
    lax.fori_loop(0, 40, zrow, 0)

    @pl.when(sid < 10)
    def _zero():
        def zblk(j, _):
            pltpu.sync_copy(rows.at[0, pl.ds(0, 40)],
                            acc.at[pl.ds(sid * 1000 + j * 40, 40)])
            return 0
        lax.fori_loop(0, 25, zblk, 0)

    pltpu.sync_copy(sidx.at[w], idx_s)
    pltpu.sync_copy(didx.at[w], idx_d)

    plsc.subcore_barrier()

    def gather_start(g, b):
        pltpu.async_copy(table.at[idx_s.at[pl.ds(g * K, K)]], rows.at[b],
                         sem_g.at[b])

    def gather_wait(g, b):
        pltpu.make_async_copy(table.at[idx_s.at[pl.ds(g * K, K)]], rows.at[b],
                              sem_g.at[b]).wait()

    def scatter_start(g, b):
        pltpu.async_copy(rows.at[b], acc.at[idx_d.at[pl.ds(g * K, K)]],
                         sem_s.at[b], add=True)

    def scatter_wait(g, b):
        pltpu.make_async_copy(rows.at[b], acc.at[idx_d.at[pl.ds(g * K, K)]],
                              sem_s.at[b]).wait()

    def step(g, b):
        gather_wait(g, b)
        scatter_start(g, b)

        @pl.when(g >= 1)
        def _ret():
            scatter_wait(g - 1, (b - 1) % _NB)

        @pl.when(g + 2 < _NCHUNK)
        def _pref():
            gather_start(g + 2, (b + 2) % _NB)

    gather_start(0, 0)
    gather_start(1, 1)

    def outer(o, _):
        for b in range(_NB):
            step(o * _NB + b, b)
        return 0

    lax.fori_loop(0, (_NCHUNK - 2) // _NB, outer, 0)
    for g in range(((_NCHUNK - 2) // _NB) * _NB, _NCHUNK):
        step(g, g % _NB)
    scatter_wait(_NCHUNK - 1, (_NCHUNK - 1) % _NB)

    plsc.subcore_barrier()

    @pl.when(sid < 10)
    def _writeout():
        pltpu.sync_copy(
            acc.at[pl.ds(sid * 1000, 1000)],
            out.at[cid, pl.ds(sid * 1000, 1000)],
        )


_BR = 2000


def _prep_body(x_ref, ds_ref, h0_ref):
    ns = lax.rsqrt(jnp.maximum(ds_ref[...], 1.0))
    h0_ref[...] = x_ref[...] * ns


def _prep(x, ds_col):
    return pl.pallas_call(
        _prep_body,
        grid=(N // _BR,),
        in_specs=[
            pl.BlockSpec((_BR, 128), lambda i: (i, 0)),
            pl.BlockSpec((_BR, 1), lambda i: (i, 0)),
        ],
        out_specs=pl.BlockSpec((_BR, 128), lambda i: (i, 0)),
        out_shape=jax.ShapeDtypeStruct((N, 128), jnp.float32),
    )(x, ds_col)


def _mid_body(p0_ref, p1_ref, dd_ref, ds_ref, w1_ref, b1_ref, w2_ref, o_ref):
    nd = lax.rsqrt(jnp.maximum(dd_ref[...], 1.0))
    ns = lax.rsqrt(jnp.maximum(ds_ref[...], 1.0))
    agg = (p0_ref[...] + p1_ref[...]) * nd
    h = jnp.dot(agg.astype(jnp.bfloat16), w1_ref[...].astype(jnp.bfloat16),
                preferred_element_type=jnp.float32)
    h = jnp.maximum(h + b1_ref[...], 0.0)
    h2 = jnp.dot(h.astype(jnp.bfloat16), w2_ref[...].astype(jnp.bfloat16),
                 preferred_element_type=jnp.float32)
    o_ref[...] = h2 * ns


def _mid(p0, p1, dd_col, ds_col, W1, b1r, W2):
    return pl.pallas_call(
        _mid_body,
        grid=(N // _BR,),
        in_specs=[
            pl.BlockSpec((_BR, 128), lambda i: (i, 0)),
            pl.BlockSpec((_BR, 128), lambda i: (i, 0)),
            pl.BlockSpec((_BR, 1), lambda i: (i, 0)),
            pl.BlockSpec((_BR, 1), lambda i: (i, 0)),
            pl.BlockSpec((128, 256), lambda i: (0, 0)),
            pl.BlockSpec((1, 256), lambda i: (0, 0)),
            pl.BlockSpec((256, 128), lambda i: (0, 0)),
        ],
        out_specs=pl.BlockSpec((_BR, 128), lambda i: (i, 0)),
        out_shape=jax.ShapeDtypeStruct((N, 128), jnp.float32),
    )(p0, p1, dd_col, ds_col, W1, b1r, W2)


def _post_body(p0_ref, p1_ref, dd_ref, b2_ref, o_ref):
    nd = lax.rsqrt(jnp.maximum(dd_ref[...], 1.0))
    agg = (p0_ref[...] + p1_ref[...]) * nd
    o_ref[...] = jnp.maximum(agg + b2_ref[...], 0.0)


def _post(p0, p1, dd_col, b2r):
    return pl.pallas_call(
        _post_body,
        grid=(N // _BR,),
        in_specs=[
            pl.BlockSpec((_BR, 128), lambda i: (i, 0)),
            pl.BlockSpec((_BR, 128), lambda i: (i, 0)),
            pl.BlockSpec((_BR, 1), lambda i: (i, 0)),
            pl.BlockSpec((1, 128), lambda i: (0, 0)),
        ],
        out_specs=pl.BlockSpec((_BR, 128), lambda i: (i, 0)),
        out_shape=jax.ShapeDtypeStruct((N, 128), jnp.float32),
    )(p0, p1, dd_col, b2r)


def kernel(x, edge_index, W1, b1, W2, b2):
    eidx = edge_index.astype(jnp.int32)
    src2 = eidx[0].reshape(NC * NS, _PER_W)
    dst3 = eidx[1].reshape(NC * NS, _PER_W)

    deg = _sc_degrees(eidx.reshape(2, NS, E // NS // K, K))
    ds_col = deg[0, 0].reshape(N, 1)
    dd_col = deg[1, 0].reshape(N, 1)

    h0 = _prep(x, ds_col)

    p = _sc_agg(h0, src2, dst3)
    h1s = _mid(p[0], p[1], dd_col, ds_col, W1, b1.reshape(1, -1), W2)

    q = _sc_agg(h1s, src2, dst3)
    return _post(q[0], q[1], dd_col, b2.reshape(1, -1))
